# Initial kernel scaffold; baseline (speedup 1.0000x reference)
import jax
import jax.numpy as jnp
from jax.experimental import pallas as pl


def kernel(hidden, router_W, router_b, shared_W1, shared_b1, shared_W2, shared_b2, routed_W1, routed_b1, routed_W2, routed_b2):
    raise NotImplementedError("write your pallas kernel here")


# speedup vs baseline: 2.2720x; 2.2720x over previous
import functools

import jax
import jax.numpy as jnp
from jax import lax
from jax.experimental import pallas as pl
from jax.experimental.pallas import tpu as pltpu
from jax.experimental.pallas import tpu_sc as plsc

_N = 4096
_D = 2048
_H = 8192
_E = 8
_K = 2
_BM = 256
_T = _N * _K // _BM + _E
_R = _T * _BM
_EPAD = 128

_NC, _NS, _NW = 2, 16, 32
_PW = _N * _K // _NW
_TW = _N // _NW
_PW2 = _N * _K // 16

_BN1 = 512
_BN2 = 512


def _router_body(x_ref, w_ref, b_ref, logits_ref, idx_ref, probs_ref):
    x = x_ref[...]
    w = w_ref[...]
    logits = jnp.dot(x, w, preferred_element_type=jnp.float32) + b_ref[...]
    logits_ref[...] = logits
    lane = lax.broadcasted_iota(jnp.int32, logits.shape, 1)
    valid = lane < _E
    lm = jnp.where(valid, logits, jnp.float32(-1e30))
    m = jnp.max(lm, axis=1, keepdims=True)
    ex = jnp.where(valid, jnp.exp(lm - m), 0.0)
    p = ex / jnp.sum(ex, axis=1, keepdims=True)
    m1 = jnp.max(p, axis=1, keepdims=True)
    i1 = jnp.min(jnp.where(p == m1, lane, _EPAD), axis=1, keepdims=True)
    p2 = jnp.where(lane == i1, -1.0, p)
    m2 = jnp.max(p2, axis=1, keepdims=True)
    i2 = jnp.min(jnp.where(p2 == m2, lane, _EPAD), axis=1, keepdims=True)
    idx_ref[...] = jnp.where(lane == 0, i1, jnp.where(lane == 1, i2, 0))
    probs_ref[...] = jnp.where(lane == 0, m1, jnp.where(lane == 1, m2, 0.0))


def _router(flat, w_pad, b_pad):
    bm = 512
    return pl.pallas_call(
        _router_body,
        grid=(_N // bm,),
        in_specs=[
            pl.BlockSpec((bm, _D), lambda i: (i, 0)),
            pl.BlockSpec((_D, _EPAD), lambda i: (0, 0)),
            pl.BlockSpec((1, _EPAD), lambda i: (0, 0)),
        ],
        out_specs=[
            pl.BlockSpec((bm, _EPAD), lambda i: (i, 0)),
            pl.BlockSpec((bm, _EPAD), lambda i: (i, 0)),
            pl.BlockSpec((bm, _EPAD), lambda i: (i, 0)),
        ],
        out_shape=[
            jax.ShapeDtypeStruct((_N, _EPAD), jnp.float32),
            jax.ShapeDtypeStruct((_N, _EPAD), jnp.int32),
            jax.ShapeDtypeStruct((_N, _EPAD), jnp.float32),
        ],
    )(flat, w_pad, b_pad)


@functools.lru_cache(maxsize=None)
def _sc_mesh():
    return plsc.VectorSubcoreMesh(core_axis_name="c", subcore_axis_name="s")


@functools.lru_cache(maxsize=None)
def _count_kernel():
  return functools.partial(
    pl.kernel,
    out_type=jax.ShapeDtypeStruct((_NW, 16), jnp.int32),
    mesh=_sc_mesh(),
    compiler_params=pltpu.CompilerParams(needs_layout_passes=False),
    scratch_types=[
        pltpu.VMEM((_PW,), jnp.int32),
        pltpu.VMEM((16,), jnp.int32),
    ],
  )(_count_body)


def _count_body(experts_hbm, cnt_hbm, ids_v, myc_v):
    wid = lax.axis_index("s") * _NC + lax.axis_index("c")
    pltpu.sync_copy(experts_hbm.at[pl.ds(wid * _PW, _PW)], ids_v)
    lanes = jnp.arange(16, dtype=jnp.int32)
    counts = jnp.zeros((16,), jnp.int32)
    for v in range(_PW // 16):
        ids = ids_v[pl.ds(v * 16, 16)]
        for e in range(_E):
            pc = jnp.sum(jnp.where(ids == e, 1, 0))
            counts = counts + jnp.where(lanes == e, pc, 0)
    myc_v[...] = counts
    pltpu.sync_copy(myc_v, cnt_hbm.at[wid])


@functools.lru_cache(maxsize=None)
def _build_kernel():
  return functools.partial(
    pl.kernel,
    out_type=[
        jax.ShapeDtypeStruct((_N * _K,), jnp.int32),
        jax.ShapeDtypeStruct((64,), jnp.int32),
    ],
    mesh=_sc_mesh(),
    compiler_params=pltpu.CompilerParams(needs_layout_passes=False),
    scratch_types=[
        pltpu.VMEM((_PW,), jnp.int32),
        pltpu.VMEM((_PW,), jnp.int32),
        pltpu.VMEM((_NW, 16), jnp.int32),
        pltpu.VMEM((64,), jnp.int32),
    ],
  )(_build_body)


def _build_body(experts_hbm, cnt_hbm, dst_hbm, te_hbm, ids_v, dst_v, call_v, te_v):
    wid = lax.axis_index("s") * _NC + lax.axis_index("c")
    base = wid * _PW
    pltpu.sync_copy(experts_hbm.at[pl.ds(base, _PW)], ids_v)
    pltpu.sync_copy(cnt_hbm, call_v)
    lanes = jnp.arange(16, dtype=jnp.int32)
    total = jnp.zeros((16,), jnp.int32)
    prefix = jnp.zeros((16,), jnp.int32)
    for w in range(_NW):
        row = call_v[w]
        total = total + row
        prefix = prefix + jnp.where(w < wid, row, 0)
    padded = ((total + (_BM - 1)) >> 8) << 8
    incl = plsc.cumsum(padded)
    excl = incl - padded
    base_vec = excl + prefix
    for v in range(_PW // 16):
        ids = ids_v[pl.ds(v * 16, 16)]
        dstv = jnp.zeros((16,), jnp.int32)
        for e in range(_E):
            m = ids == e
            mi = jnp.where(m, 1, 0)
            pc = plsc.cumsum(mi)
            be = jnp.sum(jnp.where(lanes == e, base_vec, 0))
            dstv = jnp.where(m, be + pc - 1, dstv)
            base_vec = base_vec + jnp.where(lanes == e, jnp.sum(mi), 0)
        dst_v[pl.ds(v * 16, 16)] = dstv
    pltpu.sync_copy(dst_v, dst_hbm.at[pl.ds(base, _PW)])

    @pl.when(wid == 0)
    def _():
        for j in range(4):
            s = (jnp.arange(16, dtype=jnp.int32) + j * 16) * _BM
            te = jnp.zeros((16,), jnp.int32)
            for e in range(_E):
                ie = jnp.sum(jnp.where(lanes == e, incl, 0))
                te = te + jnp.where(s >= ie, 1, 0)
            te_v[pl.ds(j * 16, 16)] = jnp.minimum(te, _E - 1)
        pltpu.sync_copy(te_v, te_hbm)


def _dispatch(experts):
    cnt = _count_kernel()(experts)
    return _build_kernel()(experts, cnt)


@functools.lru_cache(maxsize=None)
def _scatter_x_kernel():
  return functools.partial(
    pl.kernel,
    out_type=jax.ShapeDtypeStruct((_R, _D), jnp.float32),
    mesh=_sc_mesh(),
    compiler_params=pltpu.CompilerParams(needs_layout_passes=False),
    scratch_types=[
        pltpu.VMEM((16, _D), jnp.float32),
        pltpu.VMEM((_TW,), jnp.int32),
        pltpu.VMEM((_TW,), jnp.int32),
        pltpu.SemaphoreType.DMA,
        pltpu.SemaphoreType.DMA,
    ],
  )(_scatter_x_body)


def _scatter_x_body(flat_hbm, dst_hbm, xg_hbm, rows_v, da_v, db_v, sema, semb):
    wid = lax.axis_index("s") * _NC + lax.axis_index("c")
    t0 = wid * _TW
    pltpu.sync_copy(dst_hbm.at[pl.ds(t0, _TW)], da_v)
    pltpu.sync_copy(dst_hbm.at[pl.ds(_N + t0, _TW)], db_v)
    for c in range(_TW // 16):
        pltpu.sync_copy(flat_hbm.at[pl.ds(t0 + c * 16, 16)], rows_v)
        ia = da_v[pl.ds(c * 16, 16)]
        ib = db_v[pl.ds(c * 16, 16)]
        cpa = pltpu.async_copy(rows_v, xg_hbm.at[ia], sema)
        cpb = pltpu.async_copy(rows_v, xg_hbm.at[ib], semb)
        cpa.wait()
        cpb.wait()


@functools.lru_cache(maxsize=None)
def _gather_y_kernel():
  return functools.partial(
    pl.kernel,
    out_type=[
        jax.ShapeDtypeStruct((_N, _D), jnp.float32),
        jax.ShapeDtypeStruct((_N, _D), jnp.float32),
    ],
    mesh=_sc_mesh(),
    compiler_params=pltpu.CompilerParams(needs_layout_passes=False),
    scratch_types=[
        pltpu.VMEM((16, _D), jnp.float32),
        pltpu.VMEM((16, _D), jnp.float32),
        pltpu.VMEM((_TW,), jnp.int32),
        pltpu.VMEM((_TW,), jnp.int32),
        pltpu.SemaphoreType.DMA,
        pltpu.SemaphoreType.DMA,
    ],
  )(_gather_y_body)


def _gather_y_body(yg_hbm, dst_hbm, ya_hbm, yb_hbm, bufa, bufb, da_v, db_v, sema, semb):
    wid = lax.axis_index("s") * _NC + lax.axis_index("c")
    t0 = wid * _TW
    pltpu.sync_copy(dst_hbm.at[pl.ds(t0, _TW)], da_v)
    pltpu.sync_copy(dst_hbm.at[pl.ds(_N + t0, _TW)], db_v)
    for c in range(_TW // 16):
        ia = da_v[pl.ds(c * 16, 16)]
        ib = db_v[pl.ds(c * 16, 16)]
        cpa = pltpu.async_copy(yg_hbm.at[ia], bufa, sema)
        cpb = pltpu.async_copy(yg_hbm.at[ib], bufb, semb)
        cpa.wait()
        pltpu.sync_copy(bufa, ya_hbm.at[pl.ds(t0 + c * 16, 16)])
        cpb.wait()
        pltpu.sync_copy(bufb, yb_hbm.at[pl.ds(t0 + c * 16, 16)])


def _gelu(x):
    return x * 0.5 * (1.0 + lax.erf(x * 0.7071067811865476))


def _ffn1_body(em_ref, x_ref, w_ref, b_ref, o_ref):
    h = jnp.dot(x_ref[...], w_ref[0], preferred_element_type=jnp.float32)
    o_ref[...] = _gelu(h + b_ref[0])


def _ffn2_body(em_ref, x_ref, w_ref, b_ref, o_ref):
    y = jnp.dot(x_ref[...], w_ref[0], preferred_element_type=jnp.float32)
    o_ref[...] = y + b_ref[0]


def _grouped_ffn1(em, x, w1, b1, t_tiles):
    return pl.pallas_call(
        _ffn1_body,
        grid_spec=pltpu.PrefetchScalarGridSpec(
            num_scalar_prefetch=1,
            grid=(_H // _BN1, t_tiles),
            in_specs=[
                pl.BlockSpec((_BM, _D), lambda h, t, em: (t, 0)),
                pl.BlockSpec((1, _D, _BN1), lambda h, t, em: (em[t], 0, h)),
                pl.BlockSpec((1, 1, _BN1), lambda h, t, em: (em[t], 0, h)),
            ],
            out_specs=pl.BlockSpec((_BM, _BN1), lambda h, t, em: (t, h)),
        ),
        out_shape=jax.ShapeDtypeStruct((t_tiles * _BM, _H), jnp.float32),
    )(em, x, w1, b1.reshape(b1.shape[0], 1, b1.shape[1]))


def _grouped_ffn2(em, h, w2, b2, t_tiles):
    return pl.pallas_call(
        _ffn2_body,
        grid_spec=pltpu.PrefetchScalarGridSpec(
            num_scalar_prefetch=1,
            grid=(_D // _BN2, t_tiles),
            in_specs=[
                pl.BlockSpec((_BM, _H), lambda d, t, em: (t, 0)),
                pl.BlockSpec((1, _H, _BN2), lambda d, t, em: (em[t], 0, d)),
                pl.BlockSpec((1, 1, _BN2), lambda d, t, em: (em[t], 0, d)),
            ],
            out_specs=pl.BlockSpec((_BM, _BN2), lambda d, t, em: (t, d)),
        ),
        out_shape=jax.ShapeDtypeStruct((t_tiles * _BM, _D), jnp.float32),
    )(em, h, w2, b2.reshape(b2.shape[0], 1, b2.shape[1]))


def _combine_body(s_ref, ya_ref, yb_ref, pa_ref, pb_ref, o_ref):
    o_ref[...] = (s_ref[...] + pa_ref[...] * ya_ref[...]
                  + pb_ref[...] * yb_ref[...])


def _combine(shared, ya, yb, pa, pb):
    bm = 256
    return pl.pallas_call(
        _combine_body,
        grid=(_N // bm,),
        in_specs=[
            pl.BlockSpec((bm, _D), lambda i: (i, 0)),
            pl.BlockSpec((bm, _D), lambda i: (i, 0)),
            pl.BlockSpec((bm, _D), lambda i: (i, 0)),
            pl.BlockSpec((bm, 1), lambda i: (i, 0)),
            pl.BlockSpec((bm, 1), lambda i: (i, 0)),
        ],
        out_specs=pl.BlockSpec((bm, _D), lambda i: (i, 0)),
        out_shape=jax.ShapeDtypeStruct((_N, _D), jnp.float32),
    )(shared, ya, yb, pa, pb)


def kernel(hidden, router_W, router_b, shared_W1, shared_b1, shared_W2,
           shared_b2, routed_W1, routed_b1, routed_W2, routed_b2):
    batch, seq, dim = hidden.shape
    flat = hidden.reshape(batch * seq, dim)

    w_pad = jnp.pad(router_W, ((0, 0), (0, _EPAD - _E)))
    b_pad = jnp.pad(router_b, (0, _EPAD - _E)).reshape(1, _EPAD)
    logits_p, idx_p, probs_p = _router(flat, w_pad, b_pad)
    aux_logits = logits_p[:, :_E]
    idx2 = idx_p[:, :_K]
    probs2 = probs_p[:, :_K]

    experts = jnp.concatenate([idx2[:, 0], idx2[:, 1]])
    dst, te = _dispatch(experts)
    em = te[:_T]

    xg = _scatter_x_kernel()(flat, dst)
    hg = _grouped_ffn1(em, xg, routed_W1, routed_b1, _T)
    yg = _grouped_ffn2(em, hg, routed_W2, routed_b2, _T)

    em0 = jnp.zeros((_N // _BM,), jnp.int32)
    n_shared = shared_W1.shape[0]
    shared_out = jnp.zeros_like(flat)
    for i in range(n_shared):
        hs = _grouped_ffn1(em0, flat, shared_W1[i:i + 1], shared_b1[i:i + 1],
                           _N // _BM)
        shared_out = shared_out + _grouped_ffn2(
            em0, hs, shared_W2[i:i + 1], shared_b2[i:i + 1], _N // _BM)
    if n_shared > 0:
        shared_out = shared_out / n_shared

    ya, yb = _gather_y_kernel()(yg, dst)
    out = _combine(shared_out, ya, yb, probs2[:, 0:1], probs2[:, 1:2])

    return (out.reshape(batch, seq, dim),
            aux_logits.reshape(batch, seq, _E),
            idx2.reshape(batch, seq, _K),
            probs2.reshape(batch, seq, _K))


# speedup vs baseline: 2.4470x; 1.0771x over previous
import functools

import jax
import jax.numpy as jnp
from jax import lax
from jax.experimental import pallas as pl
from jax.experimental.pallas import tpu as pltpu
from jax.experimental.pallas import tpu_sc as plsc

_N = 4096
_D = 2048
_H = 8192
_E = 8
_K = 2
_BM = 256
_T = _N * _K // _BM + _E
_R = _T * _BM
_EPAD = 128

_NC, _NS, _NW = 2, 16, 32
_PW = _N * _K // _NW
_TW = _N // _NW
_PW2 = _N * _K // 16

_BN1 = 512
_BN2 = 512


def _router_body(x_ref, w_ref, b_ref, logits_ref, idx_ref, probs_ref):
    x = x_ref[...]
    w = w_ref[...]
    logits = jnp.dot(x, w, preferred_element_type=jnp.float32) + b_ref[...]
    logits_ref[...] = logits
    lane = lax.broadcasted_iota(jnp.int32, logits.shape, 1)
    valid = lane < _E
    lm = jnp.where(valid, logits, jnp.float32(-1e30))
    m = jnp.max(lm, axis=1, keepdims=True)
    ex = jnp.where(valid, jnp.exp(lm - m), 0.0)
    p = ex / jnp.sum(ex, axis=1, keepdims=True)
    m1 = jnp.max(p, axis=1, keepdims=True)
    i1 = jnp.min(jnp.where(p == m1, lane, _EPAD), axis=1, keepdims=True)
    p2 = jnp.where(lane == i1, -1.0, p)
    m2 = jnp.max(p2, axis=1, keepdims=True)
    i2 = jnp.min(jnp.where(p2 == m2, lane, _EPAD), axis=1, keepdims=True)
    idx_ref[...] = jnp.where(lane == 0, i1, jnp.where(lane == 1, i2, 0))
    probs_ref[...] = jnp.where(lane == 0, m1, jnp.where(lane == 1, m2, 0.0))


def _router(flat, w_pad, b_pad):
    bm = 512
    return pl.pallas_call(
        _router_body,
        grid=(_N // bm,),
        in_specs=[
            pl.BlockSpec((bm, _D), lambda i: (i, 0)),
            pl.BlockSpec((_D, _EPAD), lambda i: (0, 0)),
            pl.BlockSpec((1, _EPAD), lambda i: (0, 0)),
        ],
        out_specs=[
            pl.BlockSpec((bm, _EPAD), lambda i: (i, 0)),
            pl.BlockSpec((bm, _EPAD), lambda i: (i, 0)),
            pl.BlockSpec((bm, _EPAD), lambda i: (i, 0)),
        ],
        out_shape=[
            jax.ShapeDtypeStruct((_N, _EPAD), jnp.float32),
            jax.ShapeDtypeStruct((_N, _EPAD), jnp.int32),
            jax.ShapeDtypeStruct((_N, _EPAD), jnp.float32),
        ],
    )(flat, w_pad, b_pad)


@functools.lru_cache(maxsize=None)
def _sc_mesh():
    return plsc.VectorSubcoreMesh(core_axis_name="c", subcore_axis_name="s")


@functools.lru_cache(maxsize=None)
def _count_kernel():
  return functools.partial(
    pl.kernel,
    out_type=jax.ShapeDtypeStruct((_NW, 16), jnp.int32),
    mesh=_sc_mesh(),
    compiler_params=pltpu.CompilerParams(needs_layout_passes=False),
    scratch_types=[
        pltpu.VMEM((_PW,), jnp.int32),
        pltpu.VMEM((16,), jnp.int32),
    ],
  )(_count_body)


def _count_body(experts_hbm, cnt_hbm, ids_v, myc_v):
    wid = lax.axis_index("s") * _NC + lax.axis_index("c")
    pltpu.sync_copy(experts_hbm.at[pl.ds(wid * _PW, _PW)], ids_v)
    lanes = jnp.arange(16, dtype=jnp.int32)
    counts = jnp.zeros((16,), jnp.int32)
    for v in range(_PW // 16):
        ids = ids_v[pl.ds(v * 16, 16)]
        for e in range(_E):
            pc = jnp.sum(jnp.where(ids == e, 1, 0))
            counts = counts + jnp.where(lanes == e, pc, 0)
    myc_v[...] = counts
    pltpu.sync_copy(myc_v, cnt_hbm.at[wid])


@functools.lru_cache(maxsize=None)
def _build_kernel():
  return functools.partial(
    pl.kernel,
    out_type=[
        jax.ShapeDtypeStruct((_N * _K,), jnp.int32),
        jax.ShapeDtypeStruct((64,), jnp.int32),
    ],
    mesh=_sc_mesh(),
    compiler_params=pltpu.CompilerParams(needs_layout_passes=False),
    scratch_types=[
        pltpu.VMEM((_PW,), jnp.int32),
        pltpu.VMEM((_PW,), jnp.int32),
        pltpu.VMEM((_NW, 16), jnp.int32),
        pltpu.VMEM((64,), jnp.int32),
    ],
  )(_build_body)


def _build_body(experts_hbm, cnt_hbm, dst_hbm, te_hbm, ids_v, dst_v, call_v, te_v):
    wid = lax.axis_index("s") * _NC + lax.axis_index("c")
    base = wid * _PW
    pltpu.sync_copy(experts_hbm.at[pl.ds(base, _PW)], ids_v)
    pltpu.sync_copy(cnt_hbm, call_v)
    lanes = jnp.arange(16, dtype=jnp.int32)
    total = jnp.zeros((16,), jnp.int32)
    prefix = jnp.zeros((16,), jnp.int32)
    for w in range(_NW):
        row = call_v[w]
        total = total + row
        prefix = prefix + jnp.where(w < wid, row, 0)
    padded = ((total + (_BM - 1)) >> 8) << 8
    incl = plsc.cumsum(padded)
    excl = incl - padded
    base_vec = excl + prefix
    for v in range(_PW // 16):
        ids = ids_v[pl.ds(v * 16, 16)]
        dstv = jnp.zeros((16,), jnp.int32)
        for e in range(_E):
            m = ids == e
            mi = jnp.where(m, 1, 0)
            pc = plsc.cumsum(mi)
            be = jnp.sum(jnp.where(lanes == e, base_vec, 0))
            dstv = jnp.where(m, be + pc - 1, dstv)
            base_vec = base_vec + jnp.where(lanes == e, jnp.sum(mi), 0)
        dst_v[pl.ds(v * 16, 16)] = dstv
    pltpu.sync_copy(dst_v, dst_hbm.at[pl.ds(base, _PW)])

    @pl.when(wid == 0)
    def _():
        for j in range(4):
            s = (jnp.arange(16, dtype=jnp.int32) + j * 16) * _BM
            te = jnp.zeros((16,), jnp.int32)
            for e in range(_E):
                ie = jnp.sum(jnp.where(lanes == e, incl, 0))
                te = te + jnp.where(s >= ie, 1, 0)
            te_v[pl.ds(j * 16, 16)] = jnp.minimum(te, _E - 1)
        pltpu.sync_copy(te_v, te_hbm)


def _dispatch(experts):
    cnt = _count_kernel()(experts)
    return _build_kernel()(experts, cnt)


@functools.lru_cache(maxsize=None)
def _scatter_x_kernel():
  return functools.partial(
    pl.kernel,
    out_type=jax.ShapeDtypeStruct((_R, _D), jnp.float32),
    mesh=_sc_mesh(),
    compiler_params=pltpu.CompilerParams(needs_layout_passes=False),
    scratch_types=[
        pltpu.VMEM((16, _D), jnp.float32),
        pltpu.VMEM((_TW,), jnp.int32),
        pltpu.VMEM((_TW,), jnp.int32),
        pltpu.SemaphoreType.DMA,
        pltpu.SemaphoreType.DMA,
    ],
  )(_scatter_x_body)


def _scatter_x_body(flat_hbm, dst_hbm, xg_hbm, rows_v, da_v, db_v, sema, semb):
    wid = lax.axis_index("s") * _NC + lax.axis_index("c")
    t0 = wid * _TW
    pltpu.sync_copy(dst_hbm.at[pl.ds(t0, _TW)], da_v)
    pltpu.sync_copy(dst_hbm.at[pl.ds(_N + t0, _TW)], db_v)
    for c in range(_TW // 16):
        pltpu.sync_copy(flat_hbm.at[pl.ds(t0 + c * 16, 16)], rows_v)
        ia = da_v[pl.ds(c * 16, 16)]
        ib = db_v[pl.ds(c * 16, 16)]
        cpa = pltpu.async_copy(rows_v, xg_hbm.at[ia], sema)
        cpb = pltpu.async_copy(rows_v, xg_hbm.at[ib], semb)
        cpa.wait()
        cpb.wait()


@functools.lru_cache(maxsize=None)
def _gather_y_kernel():
  return functools.partial(
    pl.kernel,
    out_type=[
        jax.ShapeDtypeStruct((_N, _D), jnp.float32),
        jax.ShapeDtypeStruct((_N, _D), jnp.float32),
    ],
    mesh=_sc_mesh(),
    compiler_params=pltpu.CompilerParams(needs_layout_passes=False),
    scratch_types=[
        pltpu.VMEM((16, _D), jnp.float32),
        pltpu.VMEM((16, _D), jnp.float32),
        pltpu.VMEM((_TW,), jnp.int32),
        pltpu.VMEM((_TW,), jnp.int32),
        pltpu.SemaphoreType.DMA,
        pltpu.SemaphoreType.DMA,
    ],
  )(_gather_y_body)


def _gather_y_body(yg_hbm, dst_hbm, ya_hbm, yb_hbm, bufa, bufb, da_v, db_v, sema, semb):
    wid = lax.axis_index("s") * _NC + lax.axis_index("c")
    t0 = wid * _TW
    pltpu.sync_copy(dst_hbm.at[pl.ds(t0, _TW)], da_v)
    pltpu.sync_copy(dst_hbm.at[pl.ds(_N + t0, _TW)], db_v)
    for c in range(_TW // 16):
        ia = da_v[pl.ds(c * 16, 16)]
        ib = db_v[pl.ds(c * 16, 16)]
        cpa = pltpu.async_copy(yg_hbm.at[ia], bufa, sema)
        cpb = pltpu.async_copy(yg_hbm.at[ib], bufb, semb)
        cpa.wait()
        pltpu.sync_copy(bufa, ya_hbm.at[pl.ds(t0 + c * 16, 16)])
        cpb.wait()
        pltpu.sync_copy(bufb, yb_hbm.at[pl.ds(t0 + c * 16, 16)])


def _gelu(x):
    return x * 0.5 * (1.0 + lax.erf(x * 0.7071067811865476))


def _ffn1_body(em_ref, x_ref, w_ref, b_ref, o_ref):
    x = x_ref[...].astype(jnp.bfloat16)
    w = w_ref[0].astype(jnp.bfloat16)
    h = jnp.dot(x, w, preferred_element_type=jnp.float32)
    o_ref[...] = _gelu(h + b_ref[0]).astype(jnp.bfloat16)


def _ffn2_body(em_ref, x_ref, w_ref, b_ref, o_ref):
    w = w_ref[0].astype(jnp.bfloat16)
    y = jnp.dot(x_ref[...], w, preferred_element_type=jnp.float32)
    o_ref[...] = y + b_ref[0]


def _grouped_ffn1(em, x, w1, b1, t_tiles):
    return pl.pallas_call(
        _ffn1_body,
        grid_spec=pltpu.PrefetchScalarGridSpec(
            num_scalar_prefetch=1,
            grid=(_H // _BN1, t_tiles),
            in_specs=[
                pl.BlockSpec((_BM, _D), lambda h, t, em: (t, 0)),
                pl.BlockSpec((1, _D, _BN1), lambda h, t, em: (em[t], 0, h)),
                pl.BlockSpec((1, 1, _BN1), lambda h, t, em: (em[t], 0, h)),
            ],
            out_specs=pl.BlockSpec((_BM, _BN1), lambda h, t, em: (t, h)),
        ),
        out_shape=jax.ShapeDtypeStruct((t_tiles * _BM, _H), jnp.bfloat16),
    )(em, x, w1, b1.reshape(b1.shape[0], 1, b1.shape[1]))


def _grouped_ffn2(em, h, w2, b2, t_tiles):
    return pl.pallas_call(
        _ffn2_body,
        grid_spec=pltpu.PrefetchScalarGridSpec(
            num_scalar_prefetch=1,
            grid=(_D // _BN2, t_tiles),
            in_specs=[
                pl.BlockSpec((_BM, _H), lambda d, t, em: (t, 0)),
                pl.BlockSpec((1, _H, _BN2), lambda d, t, em: (em[t], 0, d)),
                pl.BlockSpec((1, 1, _BN2), lambda d, t, em: (em[t], 0, d)),
            ],
            out_specs=pl.BlockSpec((_BM, _BN2), lambda d, t, em: (t, d)),
        ),
        out_shape=jax.ShapeDtypeStruct((t_tiles * _BM, _D), jnp.float32),
    )(em, h, w2, b2.reshape(b2.shape[0], 1, b2.shape[1]))


def _combine_body(s_ref, ya_ref, yb_ref, pa_ref, pb_ref, o_ref):
    o_ref[...] = (s_ref[...] + pa_ref[...] * ya_ref[...]
                  + pb_ref[...] * yb_ref[...])


def _combine(shared, ya, yb, pa, pb):
    bm = 256
    return pl.pallas_call(
        _combine_body,
        grid=(_N // bm,),
        in_specs=[
            pl.BlockSpec((bm, _D), lambda i: (i, 0)),
            pl.BlockSpec((bm, _D), lambda i: (i, 0)),
            pl.BlockSpec((bm, _D), lambda i: (i, 0)),
            pl.BlockSpec((bm, 1), lambda i: (i, 0)),
            pl.BlockSpec((bm, 1), lambda i: (i, 0)),
        ],
        out_specs=pl.BlockSpec((bm, _D), lambda i: (i, 0)),
        out_shape=jax.ShapeDtypeStruct((_N, _D), jnp.float32),
    )(shared, ya, yb, pa, pb)


def kernel(hidden, router_W, router_b, shared_W1, shared_b1, shared_W2,
           shared_b2, routed_W1, routed_b1, routed_W2, routed_b2):
    batch, seq, dim = hidden.shape
    flat = hidden.reshape(batch * seq, dim)

    w_pad = jnp.pad(router_W, ((0, 0), (0, _EPAD - _E)))
    b_pad = jnp.pad(router_b, (0, _EPAD - _E)).reshape(1, _EPAD)
    logits_p, idx_p, probs_p = _router(flat, w_pad, b_pad)
    aux_logits = logits_p[:, :_E]
    idx2 = idx_p[:, :_K]
    probs2 = probs_p[:, :_K]

    experts = jnp.concatenate([idx2[:, 0], idx2[:, 1]])
    dst, te = _dispatch(experts)
    em = te[:_T]

    xg = _scatter_x_kernel()(flat, dst)
    hg = _grouped_ffn1(em, xg, routed_W1, routed_b1, _T)
    yg = _grouped_ffn2(em, hg, routed_W2, routed_b2, _T)

    em0 = jnp.zeros((_N // _BM,), jnp.int32)
    n_shared = shared_W1.shape[0]
    shared_out = jnp.zeros_like(flat)
    for i in range(n_shared):
        hs = _grouped_ffn1(em0, flat, shared_W1[i:i + 1], shared_b1[i:i + 1],
                           _N // _BM)
        shared_out = shared_out + _grouped_ffn2(
            em0, hs, shared_W2[i:i + 1], shared_b2[i:i + 1], _N // _BM)
    if n_shared > 0:
        shared_out = shared_out / n_shared

    ya, yb = _gather_y_kernel()(yg, dst)
    out = _combine(shared_out, ya, yb, probs2[:, 0:1], probs2[:, 1:2])

    return (out.reshape(batch, seq, dim),
            aux_logits.reshape(batch, seq, _E),
            idx2.reshape(batch, seq, _K),
            probs2.reshape(batch, seq, _K))
